# baseline (speedup 1.0000x reference)
import jax
import jax.numpy as jnp
from jax import lax
from jax.experimental import pallas as pl
from jax.experimental.pallas import tpu as pltpu
from jax.experimental.pallas import tpu_sc as plsc

B, L, D = 16, 2048, 1024
NS = 8
CHUNK = 1024
NJ = L // CHUNK


def _sums_body(ids_ref, x_ref, o_ref):
    j = pl.program_id(1)
    x = jnp.abs(x_ref[0])
    ids = ids_ref[0]
    iota = lax.broadcasted_iota(jnp.int32, (NS, CHUNK), 0)
    oh = (iota == ids).astype(jnp.float32)
    acc = jnp.dot(oh, x, preferred_element_type=jnp.float32)

    @pl.when(j == 0)
    def _():
        o_ref[0] = acc

    @pl.when(j != 0)
    def _():
        o_ref[0] = o_ref[0] + acc


def _segment_sums(inputs, step_ids):
    ids3 = step_ids.reshape(B * NJ, 1, CHUNK)
    return pl.pallas_call(
        _sums_body,
        grid=(B, NJ),
        in_specs=[
            pl.BlockSpec((1, 1, CHUNK), lambda b, j: (b * NJ + j, 0, 0)),
            pl.BlockSpec((1, CHUNK, D), lambda b, j: (b, j, 0)),
        ],
        out_specs=pl.BlockSpec((1, NS, D), lambda b, j: (b, 0, 0)),
        out_shape=jax.ShapeDtypeStruct((B, NS, D), jnp.float32),
    )(ids3, inputs)


def _sc_loss_body(ids_hbm, sums_hbm, lab_hbm, out_hbm,
                  ids_v, idx_v, rows_v, sortv_v, cntr_v, part_v, coll_v,
                  lab_v, out_v, shared, sem):
    c = lax.axis_index("c")
    sid = lax.axis_index("s")
    lane = lax.iota(jnp.int32, 16)
    SENT = jnp.int32(2 ** 30)

    @pl.when(c == 0)
    def _active():
        b = sid
        pltpu.sync_copy(lab_hbm, lab_v)
        pltpu.sync_copy(ids_hbm.at[pl.ds(b * L, L)], ids_v)

        def scan_body(j, carry):
            cnts, firsts = carry
            v = ids_v[pl.ds(j * 16, 16)]
            pos = j * 16 + lane
            new_c, new_f = [], []
            for s in range(1, NS):
                m = v == s
                new_c.append(cnts[s - 1]
                             + jnp.where(m, 1, 0).astype(jnp.int32))
                new_f.append(jnp.minimum(firsts[s - 1],
                                         jnp.where(m, pos, SENT)))
            return tuple(new_c), tuple(new_f)

        zero = jnp.zeros((16,), jnp.int32)
        sent = jnp.full((16,), SENT, jnp.int32)
        cnts, firsts = lax.fori_loop(
            0, L // 16, scan_body,
            (tuple(zero for _ in range(NS - 1)),
             tuple(sent for _ in range(NS - 1))))

        cnt_vec = jnp.zeros((16,), jnp.int32)
        first_vec = jnp.zeros((16,), jnp.int32)
        for s in range(1, NS):
            c_s = jnp.sum(cnts[s - 1])
            f_s = jnp.minimum(jnp.min(firsts[s - 1]), L)
            cnt_vec = cnt_vec + jnp.where(lane == s - 1, c_s, 0)
            first_vec = first_vec + jnp.where(lane == s - 1, f_s, 0)

        key = jnp.where(lane < NS - 1, first_vec * NS + lane + 1, SENT)
        vals = lane + 1
        _, sorted_vals = plsc.sort_key_val(key, vals)

        cnt_f = jnp.maximum(cnt_vec.astype(jnp.float32), 1.0)
        cntr_v[...] = 1.0 / cnt_f
        sortv_v[...] = sorted_vals
        next_vals = plsc.load_gather(sortv_v, [jnp.minimum(lane + 1, 15)])

        K = jnp.sum(jnp.where(cnt_vec > 0, 1, 0).astype(jnp.int32))
        valid = jnp.logical_and(lane + 1 < K, lane < NS - 2)
        desc = jnp.logical_and(sorted_vals > next_vals, valid)
        dcnt = jnp.sum(jnp.where(desc, 1, 0).astype(jnp.int32))

        idx_v[...] = b * NS + jnp.where(lane < NS - 1, sorted_vals, 0)
        pltpu.async_copy(sums_hbm.at[idx_v], rows_v, sem).wait()

        inv_rank = plsc.load_gather(cntr_v, [(sorted_vals - 1) & 15])
        invs = [jnp.sum(jnp.where(lane == r, inv_rank, 0.0))
                for r in range(NS - 1)]

        def e_body(j, accs):
            rows = [rows_v[r, pl.ds(j * 16, 16)] * invs[r]
                    for r in range(NS - 1)]
            new = []
            for i in range(NS - 2):
                d = jnp.maximum(rows[i] - rows[i + 1], 0.0)
                new.append(accs[i] + d * d)
            return tuple(new)

        accs = lax.fori_loop(
            0, D // 16, e_body,
            tuple(jnp.zeros((16,), jnp.float32) for _ in range(NS - 2)))

        E_vec = jnp.zeros((16,), jnp.float32)
        for i in range(NS - 2):
            e_i = jnp.sum(accs[i]) * (1.0 / D)
            E_vec = E_vec + jnp.where(lane == i, e_i, 0.0)

        valid_f = valid.astype(jnp.float32)
        desc_f = desc.astype(jnp.float32)
        tA_num = jnp.sum(E_vec * valid_f)
        tB_num = jnp.sum(jnp.maximum(1.0 - E_vec, 0.0) * desc_f)
        denA = jnp.maximum(K.astype(jnp.float32) - 1.0, 1.0)
        denB = jnp.maximum(dcnt.astype(jnp.float32), 1.0)
        num_vec = (jnp.where(lane == 0, tA_num, 0.0)
                   + jnp.where(lane == 1, tB_num, 0.0))
        den_vec = jnp.where(lane == 0, denA,
                            jnp.where(lane == 1, denB, 1.0))
        terms_vec = num_vec / den_vec

        lab_b = jnp.sum(jnp.where(lane == b, lab_v[...], 0))
        gAf = jnp.logical_and(lab_b == 1, K >= 2).astype(jnp.float32)
        gBf = jnp.logical_and(lab_b == 0, dcnt > 0).astype(jnp.float32)
        gates = (jnp.where(lane == 0, gAf, 0.0)
                 + jnp.where(lane == 1, gBf, 0.0))
        part_v[...] = (terms_vec * gates
                       + jnp.where(lane == 2, gAf + gBf, 0.0))
        pltpu.sync_copy(part_v, shared.at[b])

    plsc.subcore_barrier()

    @pl.when(jnp.logical_and(c == 0, sid == 0))
    def _finish():
        pltpu.sync_copy(shared, coll_v)
        acc = jnp.zeros((16,), jnp.float32)
        for r in range(16):
            acc = acc + coll_v[r, :]
        total = jnp.sum(jnp.where(lane <= 1, acc, 0.0))
        num = jnp.sum(jnp.where(lane == 2, acc, 0.0))
        res_vec = (jnp.where(lane == 0, total, 0.0)
                   / jnp.where(lane == 0, num + 1e-9, 1.0))
        out_v[...] = res_vec
        pltpu.sync_copy(out_v, out_hbm)


def _loss_epilogue_sc(step_ids, sums, binary_labels):
    mesh = plsc.VectorSubcoreMesh(core_axis_name="c", subcore_axis_name="s")
    f = pl.kernel(
        _sc_loss_body,
        out_type=jax.ShapeDtypeStruct((16,), jnp.float32),
        mesh=mesh,
        compiler_params=pltpu.CompilerParams(needs_layout_passes=False,
                                             use_tc_tiling_on_sc=False),
        scratch_types=[
            pltpu.VMEM((L,), jnp.int32),
            pltpu.VMEM((16,), jnp.int32),
            pltpu.VMEM((16, D), jnp.float32),
            pltpu.VMEM((16,), jnp.int32),
            pltpu.VMEM((16,), jnp.float32),
            pltpu.VMEM((16,), jnp.float32),
            pltpu.VMEM((16, 16), jnp.float32),
            pltpu.VMEM((16,), jnp.int32),
            pltpu.VMEM((16,), jnp.float32),
            pltpu.VMEM_SHARED((16, 16), jnp.float32),
            pltpu.SemaphoreType.DMA,
        ],
    )
    out = f(step_ids.reshape(B * L), sums.reshape(B * NS, D), binary_labels)
    return out[0]


def _loss_body(ids_ref, sums_ref, lab_ref, o_ref):
    ids = ids_ref[...]
    pos = lax.broadcasted_iota(jnp.int32, (B, L), 1)
    cnt_cols, first_cols = [], []
    for s in range(1, NS):
        eq = ids == s
        cnt_cols.append(jnp.sum(eq.astype(jnp.float32), axis=1, keepdims=True))
        first_cols.append(jnp.min(jnp.where(eq, pos, L), axis=1, keepdims=True))
    cnt = jnp.concatenate(cnt_cols, axis=1)
    first = jnp.concatenate(first_cols, axis=1)

    steps_row = lax.broadcasted_iota(jnp.int32, (1, NS - 1), 1) + 1
    key = first * NS + steps_row
    rank = jnp.sum((key[:, None, :] < key[:, :, None]).astype(jnp.int32),
                   axis=2)

    sums = sums_ref[...]
    means = sums[:, 1:, :] / jnp.maximum(cnt, 1.0)[:, :, None]

    Hs, vals = [], []
    for r in range(NS - 1):
        sel = (rank == r).astype(jnp.float32)
        Hs.append(jnp.sum(sel[:, :, None] * means, axis=1))
        vals.append(jnp.sum(sel * steps_row.astype(jnp.float32),
                            axis=1, keepdims=True))

    K = jnp.sum((cnt > 0).astype(jnp.int32), axis=1, keepdims=True)
    termA = jnp.zeros((B, 1), jnp.float32)
    termB = jnp.zeros((B, 1), jnp.float32)
    dcnt = jnp.zeros((B, 1), jnp.float32)
    for i in range(NS - 2):
        d = jnp.maximum(Hs[i] - Hs[i + 1], 0.0)
        E = jnp.mean(d * d, axis=1, keepdims=True)
        valid = ((i + 1) < K)
        desc = (vals[i] > vals[i + 1]) & valid
        descf = desc.astype(jnp.float32)
        dcnt = dcnt + descf
        termA = termA + E * valid.astype(jnp.float32)
        termB = termB + jnp.maximum(1.0 - E, 0.0) * descf
    termA = termA / jnp.maximum(K.astype(jnp.float32) - 1.0, 1.0)
    termB = termB / jnp.maximum(dcnt, 1.0)

    lab = lab_ref[...]
    hasA = (lab == 1) & (K >= 2)
    hasB = (lab == 0) & (dcnt > 0)
    totalb = (jnp.where(hasA, termA, 0.0) + jnp.where(hasB, termB, 0.0))
    numb = hasA.astype(jnp.float32) + hasB.astype(jnp.float32)
    total = jnp.sum(totalb)
    num = jnp.sum(numb)
    res = total / (num + 1e-9)
    o_ref[...] = jnp.full((8, 128), res, jnp.float32)


def _loss_epilogue(step_ids, sums, binary_labels):
    out = pl.pallas_call(
        _loss_body,
        out_shape=jax.ShapeDtypeStruct((8, 128), jnp.float32),
    )(step_ids, sums, binary_labels.reshape(B, 1))
    return out[0, 0]


def kernel(inputs, step_ids, binary_labels):
    sums = _segment_sums(inputs, step_ids)
    return _loss_epilogue_sc(step_ids, sums, binary_labels)


# speedup vs baseline: 1.0172x; 1.0172x over previous
import jax
import jax.numpy as jnp
from jax import lax
from jax.experimental import pallas as pl
from jax.experimental.pallas import tpu as pltpu
from jax.experimental.pallas import tpu_sc as plsc

B, L, D = 16, 2048, 1024
NS = 8
CHUNK = 1024
NJ = L // CHUNK


def _sums_body(ids_ref, x_ref, o_ref, cnt_ref, first_ref):
    j = pl.program_id(1)
    x = jnp.abs(x_ref[0])
    ids = ids_ref[0]
    iota = lax.broadcasted_iota(jnp.int32, (NS, CHUNK), 0)
    m = iota == ids
    oh = m.astype(jnp.float32)
    acc = jnp.dot(oh, x, preferred_element_type=jnp.float32)
    cnt = jnp.broadcast_to(jnp.sum(oh, axis=1, keepdims=True), (NS, 128))
    pos = j * CHUNK + lax.broadcasted_iota(jnp.int32, (NS, CHUNK), 1)
    first = jnp.broadcast_to(
        jnp.min(jnp.where(m, pos, L), axis=1, keepdims=True), (NS, 128))

    @pl.when(j == 0)
    def _():
        o_ref[0] = acc
        cnt_ref[0] = cnt
        first_ref[0] = first

    @pl.when(j != 0)
    def _():
        o_ref[0] = o_ref[0] + acc
        cnt_ref[0] = cnt_ref[0] + cnt
        first_ref[0] = jnp.minimum(first_ref[0], first)


def _segment_sums(inputs, step_ids):
    ids3 = step_ids.reshape(B * NJ, 1, CHUNK)
    return pl.pallas_call(
        _sums_body,
        grid=(B, NJ),
        in_specs=[
            pl.BlockSpec((1, 1, CHUNK), lambda b, j: (b * NJ + j, 0, 0)),
            pl.BlockSpec((1, CHUNK, D), lambda b, j: (b, j, 0)),
        ],
        out_specs=[
            pl.BlockSpec((1, NS, D), lambda b, j: (b, 0, 0)),
            pl.BlockSpec((1, NS, 128), lambda b, j: (b, 0, 0)),
            pl.BlockSpec((1, NS, 128), lambda b, j: (b, 0, 0)),
        ],
        out_shape=[
            jax.ShapeDtypeStruct((B, NS, D), jnp.float32),
            jax.ShapeDtypeStruct((B, NS, 128), jnp.float32),
            jax.ShapeDtypeStruct((B, NS, 128), jnp.int32),
        ],
    )(ids3, inputs)


def _sc_loss_body(cnt_hbm, first_hbm, sums_hbm, lab_hbm, out_hbm,
                  cnt1k_v, first1k_v, idx_v, rows_v, sortv_v, cntr_v,
                  part_v, coll_v, lab_v, out_v, shared, sem):
    c = lax.axis_index("c")
    sid = lax.axis_index("s")
    lane = lax.iota(jnp.int32, 16)
    SENT = jnp.int32(2 ** 30)

    @pl.when(c == 0)
    def _active():
        b = sid
        pltpu.sync_copy(lab_hbm, lab_v)
        pltpu.sync_copy(cnt_hbm.at[pl.ds(b * NS * 128, NS * 128)], cnt1k_v)
        pltpu.sync_copy(first_hbm.at[pl.ds(b * NS * 128, NS * 128)],
                        first1k_v)

        stepidx = jnp.minimum(lane + 1, NS - 1) * 128
        cnt_vec = jnp.where(lane < NS - 1,
                            plsc.load_gather(cnt1k_v, [stepidx]), 0.0)
        first_vec = plsc.load_gather(first1k_v, [stepidx])

        key = jnp.where(lane < NS - 1, first_vec * NS + lane + 1, SENT)
        vals = lane + 1
        _, sorted_vals = plsc.sort_key_val(key, vals)

        cnt_f = jnp.maximum(cnt_vec, 1.0)
        cntr_v[...] = 1.0 / cnt_f
        sortv_v[...] = sorted_vals
        next_vals = plsc.load_gather(sortv_v, [jnp.minimum(lane + 1, 15)])

        K = jnp.sum(jnp.where(cnt_vec > 0, 1, 0).astype(jnp.int32))
        valid = jnp.logical_and(lane + 1 < K, lane < NS - 2)
        desc = jnp.logical_and(sorted_vals > next_vals, valid)
        dcnt = jnp.sum(jnp.where(desc, 1, 0).astype(jnp.int32))

        idx_v[...] = b * NS + jnp.where(lane < NS - 1, sorted_vals, 0)
        pltpu.async_copy(sums_hbm.at[idx_v], rows_v, sem).wait()

        inv_rank = plsc.load_gather(cntr_v, [(sorted_vals - 1) & 15])
        invs = [jnp.sum(jnp.where(lane == r, inv_rank, 0.0))
                for r in range(NS - 1)]

        def e_body(j, accs):
            rows = [rows_v[r, pl.ds(j * 16, 16)] * invs[r]
                    for r in range(NS - 1)]
            new = []
            for i in range(NS - 2):
                d = jnp.maximum(rows[i] - rows[i + 1], 0.0)
                new.append(accs[i] + d * d)
            return tuple(new)

        accs = lax.fori_loop(
            0, D // 16, e_body,
            tuple(jnp.zeros((16,), jnp.float32) for _ in range(NS - 2)))

        E_vec = jnp.zeros((16,), jnp.float32)
        for i in range(NS - 2):
            e_i = jnp.sum(accs[i]) * (1.0 / D)
            E_vec = E_vec + jnp.where(lane == i, e_i, 0.0)

        valid_f = valid.astype(jnp.float32)
        desc_f = desc.astype(jnp.float32)
        tA_num = jnp.sum(E_vec * valid_f)
        tB_num = jnp.sum(jnp.maximum(1.0 - E_vec, 0.0) * desc_f)
        denA = jnp.maximum(K.astype(jnp.float32) - 1.0, 1.0)
        denB = jnp.maximum(dcnt.astype(jnp.float32), 1.0)
        num_vec = (jnp.where(lane == 0, tA_num, 0.0)
                   + jnp.where(lane == 1, tB_num, 0.0))
        den_vec = jnp.where(lane == 0, denA,
                            jnp.where(lane == 1, denB, 1.0))
        terms_vec = num_vec / den_vec

        lab_b = jnp.sum(jnp.where(lane == b, lab_v[...], 0))
        gAf = jnp.logical_and(lab_b == 1, K >= 2).astype(jnp.float32)
        gBf = jnp.logical_and(lab_b == 0, dcnt > 0).astype(jnp.float32)
        gates = (jnp.where(lane == 0, gAf, 0.0)
                 + jnp.where(lane == 1, gBf, 0.0))
        part_v[...] = (terms_vec * gates
                       + jnp.where(lane == 2, gAf + gBf, 0.0))
        pltpu.sync_copy(part_v, shared.at[b])

    plsc.subcore_barrier()

    @pl.when(jnp.logical_and(c == 0, sid == 0))
    def _finish():
        pltpu.sync_copy(shared, coll_v)
        acc = jnp.zeros((16,), jnp.float32)
        for r in range(16):
            acc = acc + coll_v[r, :]
        total = jnp.sum(jnp.where(lane <= 1, acc, 0.0))
        num = jnp.sum(jnp.where(lane == 2, acc, 0.0))
        res_vec = (jnp.where(lane == 0, total, 0.0)
                   / jnp.where(lane == 0, num + 1e-9, 1.0))
        out_v[...] = res_vec
        pltpu.sync_copy(out_v, out_hbm)


def _loss_epilogue_sc(cnt, first, sums, binary_labels):
    mesh = plsc.VectorSubcoreMesh(core_axis_name="c", subcore_axis_name="s")
    f = pl.kernel(
        _sc_loss_body,
        out_type=jax.ShapeDtypeStruct((16,), jnp.float32),
        mesh=mesh,
        compiler_params=pltpu.CompilerParams(needs_layout_passes=False,
                                             use_tc_tiling_on_sc=False),
        scratch_types=[
            pltpu.VMEM((NS * 128,), jnp.float32),
            pltpu.VMEM((NS * 128,), jnp.int32),
            pltpu.VMEM((16,), jnp.int32),
            pltpu.VMEM((16, D), jnp.float32),
            pltpu.VMEM((16,), jnp.int32),
            pltpu.VMEM((16,), jnp.float32),
            pltpu.VMEM((16,), jnp.float32),
            pltpu.VMEM((16, 16), jnp.float32),
            pltpu.VMEM((16,), jnp.int32),
            pltpu.VMEM((16,), jnp.float32),
            pltpu.VMEM_SHARED((16, 16), jnp.float32),
            pltpu.SemaphoreType.DMA,
        ],
    )
    out = f(cnt.reshape(B * NS * 128), first.reshape(B * NS * 128),
            sums.reshape(B * NS, D), binary_labels)
    return out[0]


def kernel(inputs, step_ids, binary_labels):
    sums, cnt, first = _segment_sums(inputs, step_ids)
    return _loss_epilogue_sc(cnt, first, sums, binary_labels)
